# baseline (device time: 80806 ns/iter reference)
import jax
import jax.numpy as jnp
from jax import lax
from jax.experimental import pallas as pl
from jax.experimental.pallas import tpu as pltpu

N_DEV = 8
E_PER = 4


def kernel(x, router_W, route_idx, expert_W):
    n, d = x.shape
    e_loc, _, h = expert_W.shape
    chunk = n // N_DEV
    n_hops = N_DEV - 1

    def body(x_ref, idx_ref, w_ref, out_ref,
             partial_ref, rs_send_ref, rs_recv_ref, ag_recv_ref,
             send_sems, recv_sems):
        me = lax.axis_index("i")
        left = lax.rem(me - 1 + N_DEV, N_DEV)
        right = lax.rem(me + 1, N_DEV)

        barrier_sem = pltpu.get_barrier_semaphore()
        for nbr in (left, right):
            pl.semaphore_signal(
                barrier_sem, inc=1,
                device_id=(nbr,), device_id_type=pl.DeviceIdType.MESH,
            )
        pl.semaphore_wait(barrier_sem, 2)

        xb = x_ref[...].astype(jnp.bfloat16)
        acc = jnp.zeros((n, h), jnp.float32)
        for k in range(E_PER):
            eid = me * E_PER + k
            mask = idx_ref[...] == eid
            xk = jnp.where(mask, xb, jnp.zeros_like(xb))
            wk = w_ref[k].astype(jnp.bfloat16)
            acc = acc + jnp.dot(xk, wk, preferred_element_type=jnp.float32)
        partial_ref[...] = acc.astype(jnp.bfloat16)

        for s in range(n_hops):
            c = lax.rem(me - s + N_DEV, N_DEV)
            val = partial_ref[pl.ds(c * chunk, chunk), :]
            if s > 0:
                val = val + rs_recv_ref[s - 1]
            rs_send_ref[...] = val
            rdma = pltpu.make_async_remote_copy(
                src_ref=rs_send_ref,
                dst_ref=rs_recv_ref.at[s],
                send_sem=send_sems.at[s],
                recv_sem=recv_sems.at[s],
                device_id=(right,),
                device_id_type=pl.DeviceIdType.MESH,
            )
            rdma.start()
            rdma.wait()

        own = lax.rem(me + 1, N_DEV)
        own_val = (
            partial_ref[pl.ds(own * chunk, chunk), :] + rs_recv_ref[n_hops - 1]
        )
        rs_send_ref[...] = own_val
        out_ref[pl.ds(own * chunk, chunk), :] = own_val.astype(jnp.float32)

        for g in range(n_hops):
            src = rs_send_ref if g == 0 else ag_recv_ref.at[g - 1]
            rdma = pltpu.make_async_remote_copy(
                src_ref=src,
                dst_ref=ag_recv_ref.at[g],
                send_sem=send_sems.at[n_hops + g],
                recv_sem=recv_sems.at[n_hops + g],
                device_id=(right,),
                device_id_type=pl.DeviceIdType.MESH,
            )
            rdma.start()
            rdma.wait()
            c = lax.rem(me - g + N_DEV, N_DEV)
            out_ref[pl.ds(c * chunk, chunk), :] = (
                ag_recv_ref[g].astype(jnp.float32)
            )

    return pl.pallas_call(
        body,
        out_shape=jax.ShapeDtypeStruct((n, h), jnp.float32),
        in_specs=[
            pl.BlockSpec(memory_space=pltpu.VMEM),
            pl.BlockSpec(memory_space=pltpu.VMEM),
            pl.BlockSpec(memory_space=pltpu.VMEM),
        ],
        out_specs=pl.BlockSpec(memory_space=pltpu.VMEM),
        scratch_shapes=[
            pltpu.VMEM((n, h), jnp.bfloat16),
            pltpu.VMEM((chunk, h), jnp.bfloat16),
            pltpu.VMEM((n_hops, chunk, h), jnp.bfloat16),
            pltpu.VMEM((n_hops, chunk, h), jnp.bfloat16),
            pltpu.SemaphoreType.DMA((2 * n_hops,)),
            pltpu.SemaphoreType.DMA((2 * n_hops,)),
        ],
        compiler_params=pltpu.CompilerParams(collective_id=0),
    )(x, route_idx, expert_W)


# device time: 44458 ns/iter; 1.8176x vs baseline; 1.8176x over previous
import jax
import jax.numpy as jnp
from jax import lax
from jax.experimental import pallas as pl
from jax.experimental.pallas import tpu as pltpu

N_DEV = 8
E_PER = 4


def kernel(x, router_W, route_idx, expert_W):
    n, d = x.shape
    e_loc, _, h = expert_W.shape
    chunk = n // N_DEV

    def body(x_ref, idx_ref, w_ref, out_ref,
             wbf_ref, pchunk_ref, rs_recv_ref, ag_src_ref, ag_recv_ref,
             rs_send_sems, rs_recv_sems, ag_send_sems, ag_recv_sems):
        me = lax.axis_index("i")

        barrier_sem = pltpu.get_barrier_semaphore()
        for j in range(1, N_DEV):
            peer = lax.rem(me + j, N_DEV)
            pl.semaphore_signal(
                barrier_sem, inc=1,
                device_id=(peer,), device_id_type=pl.DeviceIdType.MESH,
            )
        pl.semaphore_wait(barrier_sem, N_DEV - 1)

        wbf_ref[...] = w_ref[...].astype(jnp.bfloat16)

        def chunk_partial(c):
            xb = x_ref[pl.ds(c * chunk, chunk), :].astype(jnp.bfloat16)
            idc = idx_ref[pl.ds(c * chunk, chunk), :]
            acc = jnp.zeros((chunk, h), jnp.float32)
            for k in range(E_PER):
                eid = me * E_PER + k
                xk = jnp.where(idc == eid, xb, jnp.zeros_like(xb))
                acc = acc + jnp.dot(
                    xk, wbf_ref[k], preferred_element_type=jnp.float32
                )
            return acc.astype(jnp.bfloat16)

        rs_sends = []
        for j in range(1, N_DEV):
            p = lax.rem(me + j, N_DEV)
            pchunk_ref[j, :, :] = chunk_partial(p)
            rdma = pltpu.make_async_remote_copy(
                src_ref=pchunk_ref.at[j],
                dst_ref=rs_recv_ref.at[N_DEV - j],
                send_sem=rs_send_sems.at[j],
                recv_sem=rs_recv_sems.at[N_DEV - j],
                device_id=(p,),
                device_id_type=pl.DeviceIdType.MESH,
            )
            rdma.start()
            rs_sends.append(rdma)

        own = chunk_partial(me)

        for k in range(1, N_DEV):
            rs_sends[k - 1].wait_recv()
        for k in range(1, N_DEV):
            own = own + rs_recv_ref[k]
        ag_src_ref[...] = own
        out_ref[pl.ds(me * chunk, chunk), :] = own.astype(jnp.float32)

        ag_sends = []
        for j in range(1, N_DEV):
            p = lax.rem(me + j, N_DEV)
            rdma = pltpu.make_async_remote_copy(
                src_ref=ag_src_ref,
                dst_ref=ag_recv_ref.at[N_DEV - j],
                send_sem=ag_send_sems.at[j],
                recv_sem=ag_recv_sems.at[N_DEV - j],
                device_id=(p,),
                device_id_type=pl.DeviceIdType.MESH,
            )
            rdma.start()
            ag_sends.append(rdma)

        for k in range(N_DEV - 1, 0, -1):
            ag_sends[(N_DEV - k) - 1].wait_recv()
            c = lax.rem(me + k, N_DEV)
            out_ref[pl.ds(c * chunk, chunk), :] = (
                ag_recv_ref[k].astype(jnp.float32)
            )

        for r in rs_sends:
            r.wait_send()
        for r in ag_sends:
            r.wait_send()

    return pl.pallas_call(
        body,
        out_shape=jax.ShapeDtypeStruct((n, h), jnp.float32),
        in_specs=[
            pl.BlockSpec(memory_space=pltpu.VMEM),
            pl.BlockSpec(memory_space=pltpu.VMEM),
            pl.BlockSpec(memory_space=pltpu.VMEM),
        ],
        out_specs=pl.BlockSpec(memory_space=pltpu.VMEM),
        scratch_shapes=[
            pltpu.VMEM((e_loc, d, h), jnp.bfloat16),
            pltpu.VMEM((N_DEV, chunk, h), jnp.bfloat16),
            pltpu.VMEM((N_DEV, chunk, h), jnp.bfloat16),
            pltpu.VMEM((chunk, h), jnp.bfloat16),
            pltpu.VMEM((N_DEV, chunk, h), jnp.bfloat16),
            pltpu.SemaphoreType.DMA((N_DEV,)),
            pltpu.SemaphoreType.DMA((N_DEV,)),
            pltpu.SemaphoreType.DMA((N_DEV,)),
            pltpu.SemaphoreType.DMA((N_DEV,)),
        ],
        compiler_params=pltpu.CompilerParams(collective_id=0),
    )(x, route_idx, expert_W)
